# baseline (speedup 1.0000x reference)
import functools

import jax
import jax.numpy as jnp
from jax import lax
from jax.experimental import pallas as pl
from jax.experimental.pallas import tpu as pltpu
from jax.experimental.pallas import tpu_sc as plsc

MEMORY_UNIT = 256
MAX_MEMORY = 100000
OUT_DIM = 512
UPDATE_SIZE = 3 + MEMORY_UNIT
Y_DIM = OUT_DIM - UPDATE_SIZE
JUMP_THRESHOLD = 0.5
MIN_SIM_TO_JUMP = 0.5

CHUNK = 160
NUM_CHUNKS = MAX_MEMORY // CHUNK
NW = 32
L = 16


def _controller_kernel(xj_ref, w_mat_ref, b_ref, out_ref):
    out_ref[...] = (
        jax.lax.dot_general(
            xj_ref[...], w_mat_ref[...], (((1,), (1,)), ((), ())),
            preferred_element_type=jnp.float32,
            precision=jax.lax.Precision.HIGHEST,
        )
        + b_ref[...]
    )


def _sc_scan(mem_hbm, m_hbm, sjw_hbm, mem2_hbm, d2_hbm,
             buf_v, d2_v, m_v, sjw_v, sem):
    wid = lax.axis_index("s") * 2 + lax.axis_index("c")
    pltpu.sync_copy(m_hbm, m_v.at[pl.ds(0, MEMORY_UNIT)])
    pltpu.sync_copy(sjw_hbm, sjw_v)
    w = sjw_v[...][15]
    nk = jnp.where(wid < NUM_CHUNKS % NW, NUM_CHUNKS // NW + 1,
                   NUM_CHUNKS // NW)

    def chunk_body(k, carry):
        cid = wid + NW * k
        base = cid * CHUNK
        pltpu.sync_copy(mem_hbm.at[pl.ds(base, CHUNK)], buf_v)

        @pl.when((cid == 0) & (w > 0.5))
        def _():
            for c in range(MEMORY_UNIT // L):
                buf_v[0, pl.ds(c * L, L)] = m_v[pl.ds(c * L, L)]

        pltpu.sync_copy(buf_v, mem2_hbm.at[pl.ds(base, CHUNK)])

        iota = lax.iota(jnp.int32, L)
        for g in range(CHUNK // L):
            row_idx = iota + g * L
            zero = jnp.zeros((L,), jnp.float32)

            def col_body(c0, accs):
                a0, a1, a2, a3 = accs
                res = [a0, a1, a2, a3]
                mseg = m_v[pl.ds(c0 * 8, L)]
                for u in range(8):
                    c = c0 * 8 + u
                    col_idx = jnp.full((L,), c, jnp.int32)
                    v = plsc.load_gather(buf_v, [row_idx, col_idx])
                    d = v - mseg[u]
                    res[u % 4] = res[u % 4] + d * d
                return tuple(res)

            a0, a1, a2, a3 = lax.fori_loop(
                0, MEMORY_UNIT // 8, col_body, (zero, zero, zero, zero))
            d2_v[pl.ds(g * L, L)] = (a0 + a1) + (a2 + a3)

        pltpu.sync_copy(d2_v, d2_hbm.at[pl.ds(base, CHUNK)])
        return carry

    lax.fori_loop(0, nk, chunk_body, 0)


def _epilogue_kernel(ctrl_ref, d2_ref, mem2_any_ref, read_ref,
                     land_ref, sem):
    d2 = d2_ref[...]
    sims = 1.0 - jnp.sqrt(d2) * (1.0 / MEMORY_UNIT)
    best = jnp.max(sims)
    r = jax.lax.broadcasted_iota(jnp.int32, sims.shape, 0)
    c = jax.lax.broadcasted_iota(jnp.int32, sims.shape, 1)
    flat = r * sims.shape[1] + c
    pos = jnp.min(jnp.where(sims == best, flat, MAX_MEMORY))

    s = ctrl_ref[0, Y_DIM]
    j = ctrl_ref[0, Y_DIM + 1]
    jumped = jnp.where(best > MIN_SIM_TO_JUMP, pos, 0)
    head0 = jnp.where(j > JUMP_THRESHOLD, jumped, 0)
    shift = jnp.floor(s * 3.0 - 1e-9).astype(jnp.int32) - 1
    head = jnp.mod(head0 + shift, MAX_MEMORY)
    copy = pltpu.make_async_copy(
        mem2_any_ref.at[pl.ds(head, 1)], land_ref, sem)
    copy.start()
    copy.wait()
    read_ref[...] = land_ref[...]


def kernel(x, W, b, memory, previous_read, interpret=False):
    xj = jnp.concatenate([x, previous_read[None, :]], axis=1)

    out = pl.pallas_call(
        _controller_kernel,
        out_shape=jax.ShapeDtypeStruct((1, OUT_DIM), jnp.float32),
        interpret=interpret,
    )(xj, W, b[None, :])

    y = out[0, :Y_DIM]
    m_vec = out[0, Y_DIM + 3:]
    sjw = out[0, Y_DIM - 13:Y_DIM + 3]

    sc_fn = pl.kernel(
        _sc_scan,
        mesh=plsc.VectorSubcoreMesh(core_axis_name="c", subcore_axis_name="s"),
        out_type=[
            jax.ShapeDtypeStruct((MAX_MEMORY, MEMORY_UNIT), jnp.float32),
            jax.ShapeDtypeStruct((MAX_MEMORY,), jnp.float32),
        ],
        scratch_types=[
            pltpu.VMEM((CHUNK, MEMORY_UNIT), jnp.float32),
            pltpu.VMEM((CHUNK,), jnp.float32),
            pltpu.VMEM((MEMORY_UNIT + L,), jnp.float32),
            pltpu.VMEM((L,), jnp.float32),
            pltpu.SemaphoreType.DMA,
        ],
        compiler_params=pltpu.CompilerParams(needs_layout_passes=False),
    )
    mem2, d2 = sc_fn(memory, m_vec, sjw)

    read = pl.pallas_call(
        _epilogue_kernel,
        in_specs=[
            pl.BlockSpec((1, OUT_DIM), lambda: (0, 0)),
            pl.BlockSpec((MAX_MEMORY // 125, 125), lambda: (0, 0)),
            pl.BlockSpec(memory_space=pl.ANY),
        ],
        out_specs=pl.BlockSpec((1, MEMORY_UNIT), lambda: (0, 0)),
        out_shape=jax.ShapeDtypeStruct((1, MEMORY_UNIT), jnp.float32),
        scratch_shapes=[
            pltpu.VMEM((1, MEMORY_UNIT), jnp.float32),
            pltpu.SemaphoreType.DMA,
        ],
        interpret=interpret,
    )(out, d2.reshape(MAX_MEMORY // 125, 125), mem2)

    return y, read[0], mem2
